# baseline (speedup 1.0000x reference)
import functools

import jax
import jax.numpy as jnp
from jax import lax
from jax.experimental import pallas as pl
from jax.experimental.pallas import tpu as pltpu
from jax.experimental.pallas import tpu_sc as plsc

N = 10000
E = 320000
NODEDIM = 128
RELDIM = 16
HDIM = 128

NC = 2
NS = 16
NW = NC * NS
EW = E // NW
C = 80
NCHUNK = EW // C
NPAD = 10240
NSTR = NPAD // NS

_MESH = dict(core_axis_name="c", subcore_axis_name="s",
             num_cores=NC, num_subcores=NS)


@functools.cache
def _sc_gather_kernel():
    @functools.partial(
        pl.kernel,
        out_type=jax.ShapeDtypeStruct((E, HDIM), jnp.float32),
        mesh=plsc.VectorSubcoreMesh(**_MESH),
        scratch_types=[
            pltpu.VMEM((NCHUNK, C), jnp.int32),
            pltpu.VMEM((C, HDIM), jnp.float32),
            pltpu.SemaphoreType.DMA,
        ],
    )
    def _sc_gather(h_hbm, src_hbm, out_hbm, idx_v, buf, sem):
        wid = lax.axis_index("s") * NC + lax.axis_index("c")
        pltpu.sync_copy(src_hbm.at[wid], idx_v)

        def body(ci, carry):
            pltpu.async_copy(h_hbm.at[idx_v.at[ci]], buf, sem).wait()
            pltpu.sync_copy(buf, out_hbm.at[pl.ds(wid * EW + ci * C, C)])
            return carry

        lax.fori_loop(0, NCHUNK, body, 0)

    return _sc_gather


@functools.cache
def _sc_scatter_kernel():
    @functools.partial(
        pl.kernel,
        out_type=jax.ShapeDtypeStruct((NC, NPAD, HDIM), jnp.float32),
        mesh=plsc.VectorSubcoreMesh(**_MESH),
        scratch_types=[
            pltpu.VMEM((NCHUNK, C), jnp.int32),
            pltpu.VMEM((C, HDIM), jnp.float32),
            pltpu.VMEM_SHARED((NPAD, HDIM), jnp.float32),
            pltpu.SemaphoreType.DMA,
        ],
    )
    def _sc_scatter(msg_hbm, dst_hbm, zeros_hbm, out_hbm,
                    idx_v, buf, acc_sh, sem):
        cid = lax.axis_index("c")
        sid = lax.axis_index("s")
        wid = sid * NC + cid
        pltpu.sync_copy(zeros_hbm, acc_sh.at[pl.ds(sid * NSTR, NSTR)])
        plsc.subcore_barrier()
        pltpu.sync_copy(dst_hbm.at[wid], idx_v)

        def body(ci, carry):
            pltpu.async_copy(msg_hbm.at[pl.ds(wid * EW + ci * C, C)],
                             buf, sem).wait()
            pltpu.sync_copy(buf, acc_sh.at[idx_v.at[ci]], add=True)
            return carry

        lax.fori_loop(0, NCHUNK, body, 0)
        plsc.subcore_barrier()
        pltpu.sync_copy(acc_sh.at[pl.ds(sid * NSTR, NSTR)],
                        out_hbm.at[cid, pl.ds(sid * NSTR, NSTR)])

    return _sc_scatter


def _edge_body(ea_ref, hs_ref, wi_ref, whh_ref, bi_ref, bh_ref, out_ref):
    gi = jnp.dot(ea_ref[...], wi_ref[...],
                 preferred_element_type=jnp.float32) + bi_ref[...]
    gh = jnp.dot(hs_ref[...].astype(jnp.bfloat16),
                 whh_ref[...].astype(jnp.bfloat16),
                 preferred_element_type=jnp.float32) + bh_ref[...]
    r = jax.nn.sigmoid(gi[:, :HDIM] + gh[:, :HDIM])
    z = jax.nn.sigmoid(gi[:, HDIM:2 * HDIM] + gh[:, HDIM:2 * HDIM])
    n = jnp.tanh(gi[:, 2 * HDIM:] + r * gh[:, 2 * HDIM:])
    out_ref[...] = (1.0 - z) * n + z * hs_ref[...]


BE = 512
GE = E // BE


def _tc_edge(edge_attr, h_src, wi, whh, bi, bh):
    return pl.pallas_call(
        _edge_body,
        grid=(GE,),
        in_specs=[
            pl.BlockSpec((BE, RELDIM), lambda i: (i, 0)),
            pl.BlockSpec((BE, HDIM), lambda i: (i, 0)),
            pl.BlockSpec((RELDIM, 3 * HDIM), lambda i: (0, 0)),
            pl.BlockSpec((HDIM, 3 * HDIM), lambda i: (0, 0)),
            pl.BlockSpec((1, 3 * HDIM), lambda i: (0, 0)),
            pl.BlockSpec((1, 3 * HDIM), lambda i: (0, 0)),
        ],
        out_specs=pl.BlockSpec((BE, HDIM), lambda i: (i, 0)),
        out_shape=jax.ShapeDtypeStruct((E, HDIM), jnp.float32),
    )(edge_attr, h_src, wi, whh, bi, bh)


def _node_body(x_ref, parts_ref, wi_ref, whh_ref, bi_ref, bh_ref, out_ref):
    red = parts_ref[0] + parts_ref[1]
    gi = jnp.dot(x_ref[...], wi_ref[...],
                 preferred_element_type=jnp.float32) + bi_ref[...]
    gh = jnp.dot(red, whh_ref[...],
                 preferred_element_type=jnp.float32) + bh_ref[...]
    r = jax.nn.sigmoid(gi[:, :HDIM] + gh[:, :HDIM])
    z = jax.nn.sigmoid(gi[:, HDIM:2 * HDIM] + gh[:, HDIM:2 * HDIM])
    n = jnp.tanh(gi[:, 2 * HDIM:] + r * gh[:, 2 * HDIM:])
    out_ref[...] = (1.0 - z) * n + z * red


BN = 1000
GN = N // BN


def _tc_node(x, parts, wi, whh, bi, bh):
    return pl.pallas_call(
        _node_body,
        grid=(GN,),
        in_specs=[
            pl.BlockSpec((BN, NODEDIM), lambda i: (i, 0)),
            pl.BlockSpec((NC, BN, HDIM), lambda i: (0, i, 0)),
            pl.BlockSpec((NODEDIM, 3 * HDIM), lambda i: (0, 0)),
            pl.BlockSpec((HDIM, 3 * HDIM), lambda i: (0, 0)),
            pl.BlockSpec((1, 3 * HDIM), lambda i: (0, 0)),
            pl.BlockSpec((1, 3 * HDIM), lambda i: (0, 0)),
        ],
        out_specs=pl.BlockSpec((BN, HDIM), lambda i: (i, 0)),
        out_shape=jax.ShapeDtypeStruct((N, HDIM), jnp.float32),
    )(x, parts, wi, whh, bi, bh)


def kernel(x, h, edge_index, edge_attr, W_ih_rel, W_hh_rel, b_ih_rel,
           b_hh_rel, W_ih_node, W_hh_node, b_ih_node, b_hh_node):
    src = edge_index[0].reshape(NW, NCHUNK, C)
    dst = edge_index[1].reshape(NW, NCHUNK, C)
    zeros = jnp.zeros((NSTR, HDIM), jnp.float32)

    h_src = _sc_gather_kernel()(h, src)
    msg = _tc_edge(edge_attr, h_src,
                   W_ih_rel.T, W_hh_rel.T,
                   b_ih_rel.reshape(1, -1), b_hh_rel.reshape(1, -1))
    parts = _sc_scatter_kernel()(msg, dst, zeros)[:, :N]
    h_new = _tc_node(x, parts,
                     W_ih_node.T, W_hh_node.T,
                     b_ih_node.reshape(1, -1), b_hh_node.reshape(1, -1))
    return h_new


# speedup vs baseline: 1.3091x; 1.3091x over previous
import functools

import jax
import jax.numpy as jnp
from jax import lax
from jax.experimental import pallas as pl
from jax.experimental.pallas import tpu as pltpu
from jax.experimental.pallas import tpu_sc as plsc

N = 10000
E = 320000
NODEDIM = 128
RELDIM = 16
HDIM = 128

NC = 2
NS = 16
NW = NC * NS
EW = E // NW
C = 80
NCHUNK = EW // C
NPAD = 10240
NSTR = NPAD // NS

_MESH = dict(core_axis_name="c", subcore_axis_name="s",
             num_cores=NC, num_subcores=NS)


@functools.cache
def _sc_gather_kernel():
    @functools.partial(
        pl.kernel,
        out_type=jax.ShapeDtypeStruct((E, HDIM), jnp.float32),
        mesh=plsc.VectorSubcoreMesh(**_MESH),
        scratch_types=[
            pltpu.VMEM((NCHUNK, C), jnp.int32),
            pltpu.VMEM((C, HDIM), jnp.float32),
            pltpu.SemaphoreType.DMA,
        ],
    )
    def _sc_gather(h_hbm, src_hbm, out_hbm, idx_v, buf, sem):
        wid = lax.axis_index("s") * NC + lax.axis_index("c")
        pltpu.sync_copy(src_hbm.at[wid], idx_v)

        def body(ci, carry):
            pltpu.async_copy(h_hbm.at[idx_v.at[ci]], buf, sem).wait()
            pltpu.sync_copy(buf, out_hbm.at[pl.ds(wid * EW + ci * C, C)])
            return carry

        lax.fori_loop(0, NCHUNK, body, 0)

    return _sc_gather


@functools.cache
def _sc_scatter_kernel():
    @functools.partial(
        pl.kernel,
        out_type=jax.ShapeDtypeStruct((NC, NPAD, HDIM), jnp.float32),
        mesh=plsc.VectorSubcoreMesh(**_MESH),
        scratch_types=[
            pltpu.VMEM((NCHUNK, C), jnp.int32),
            pltpu.VMEM((C, HDIM), jnp.float32),
            pltpu.VMEM_SHARED((NPAD, HDIM), jnp.float32),
            pltpu.SemaphoreType.DMA,
        ],
    )
    def _sc_scatter(msg_hbm, dst_hbm, zeros_hbm, out_hbm,
                    idx_v, buf, acc_sh, sem):
        cid = lax.axis_index("c")
        sid = lax.axis_index("s")
        wid = sid * NC + cid
        pltpu.sync_copy(zeros_hbm, acc_sh.at[pl.ds(sid * NSTR, NSTR)])
        plsc.subcore_barrier()
        pltpu.sync_copy(dst_hbm.at[wid], idx_v)

        def body(ci, carry):
            pltpu.async_copy(msg_hbm.at[pl.ds(wid * EW + ci * C, C)],
                             buf, sem).wait()
            pltpu.sync_copy(buf, acc_sh.at[idx_v.at[ci]], add=True)
            return carry

        lax.fori_loop(0, NCHUNK, body, 0)
        plsc.subcore_barrier()
        pltpu.sync_copy(acc_sh.at[pl.ds(sid * NSTR, NSTR)],
                        out_hbm.at[cid, pl.ds(sid * NSTR, NSTR)])

    return _sc_scatter


def _edge_body(ea_ref, hs_ref, wi_ref, whh_ref, bi_ref, bh_ref, out_ref):
    gi = jnp.dot(ea_ref[...], wi_ref[...],
                 preferred_element_type=jnp.float32) + bi_ref[...]
    gh = jnp.dot(hs_ref[...].astype(jnp.bfloat16),
                 whh_ref[...].astype(jnp.bfloat16),
                 preferred_element_type=jnp.float32) + bh_ref[...]
    r = jax.nn.sigmoid(gi[:, :HDIM] + gh[:, :HDIM])
    z = jax.nn.sigmoid(gi[:, HDIM:2 * HDIM] + gh[:, HDIM:2 * HDIM])
    n = jnp.tanh(gi[:, 2 * HDIM:] + r * gh[:, 2 * HDIM:])
    out_ref[...] = (1.0 - z) * n + z * hs_ref[...]


BE = 2000
GE = E // BE


def _tc_edge(edge_attr, h_src, wi, whh, bi, bh):
    return pl.pallas_call(
        _edge_body,
        grid=(GE,),
        in_specs=[
            pl.BlockSpec((BE, RELDIM), lambda i: (i, 0)),
            pl.BlockSpec((BE, HDIM), lambda i: (i, 0)),
            pl.BlockSpec((RELDIM, 3 * HDIM), lambda i: (0, 0)),
            pl.BlockSpec((HDIM, 3 * HDIM), lambda i: (0, 0)),
            pl.BlockSpec((1, 3 * HDIM), lambda i: (0, 0)),
            pl.BlockSpec((1, 3 * HDIM), lambda i: (0, 0)),
        ],
        out_specs=pl.BlockSpec((BE, HDIM), lambda i: (i, 0)),
        out_shape=jax.ShapeDtypeStruct((E, HDIM), jnp.float32),
    )(edge_attr, h_src, wi, whh, bi, bh)


def _node_body(x_ref, parts_ref, wi_ref, whh_ref, bi_ref, bh_ref, out_ref):
    red = parts_ref[0] + parts_ref[1]
    gi = jnp.dot(x_ref[...], wi_ref[...],
                 preferred_element_type=jnp.float32) + bi_ref[...]
    gh = jnp.dot(red, whh_ref[...],
                 preferred_element_type=jnp.float32) + bh_ref[...]
    r = jax.nn.sigmoid(gi[:, :HDIM] + gh[:, :HDIM])
    z = jax.nn.sigmoid(gi[:, HDIM:2 * HDIM] + gh[:, HDIM:2 * HDIM])
    n = jnp.tanh(gi[:, 2 * HDIM:] + r * gh[:, 2 * HDIM:])
    out_ref[...] = (1.0 - z) * n + z * red


BN = 1000
GN = N // BN


def _tc_node(x, parts, wi, whh, bi, bh):
    return pl.pallas_call(
        _node_body,
        grid=(GN,),
        in_specs=[
            pl.BlockSpec((BN, NODEDIM), lambda i: (i, 0)),
            pl.BlockSpec((NC, BN, HDIM), lambda i: (0, i, 0)),
            pl.BlockSpec((NODEDIM, 3 * HDIM), lambda i: (0, 0)),
            pl.BlockSpec((HDIM, 3 * HDIM), lambda i: (0, 0)),
            pl.BlockSpec((1, 3 * HDIM), lambda i: (0, 0)),
            pl.BlockSpec((1, 3 * HDIM), lambda i: (0, 0)),
        ],
        out_specs=pl.BlockSpec((BN, HDIM), lambda i: (i, 0)),
        out_shape=jax.ShapeDtypeStruct((N, HDIM), jnp.float32),
    )(x, parts, wi, whh, bi, bh)


def kernel(x, h, edge_index, edge_attr, W_ih_rel, W_hh_rel, b_ih_rel,
           b_hh_rel, W_ih_node, W_hh_node, b_ih_node, b_hh_node):
    src = edge_index[0].reshape(NW, NCHUNK, C)
    dst = edge_index[1].reshape(NW, NCHUNK, C)
    zeros = jnp.zeros((NSTR, HDIM), jnp.float32)

    h_src = _sc_gather_kernel()(h, src)
    msg = _tc_edge(edge_attr, h_src,
                   W_ih_rel.T, W_hh_rel.T,
                   b_ih_rel.reshape(1, -1), b_hh_rel.reshape(1, -1))
    parts = _sc_scatter_kernel()(msg, dst, zeros)[:, :N]
    h_new = _tc_node(x, parts,
                     W_ih_node.T, W_hh_node.T,
                     b_ih_node.reshape(1, -1), b_hh_node.reshape(1, -1))
    return h_new


# speedup vs baseline: 1.7563x; 1.3416x over previous
import functools

import jax
import jax.numpy as jnp
from jax import lax
from jax.experimental import pallas as pl
from jax.experimental.pallas import tpu as pltpu
from jax.experimental.pallas import tpu_sc as plsc

N = 10000
E = 320000
NODEDIM = 128
RELDIM = 16
HDIM = 128

NC = 2
NS = 16
NW = NC * NS
NSLICE = 5
ES = E // NSLICE
EW = ES // NW
C = 80
NCHUNK = EW // C
NPAD = 10240
NSTR = NPAD // NS

_MESH = dict(core_axis_name="c", subcore_axis_name="s",
             num_cores=NC, num_subcores=NS)


@functools.cache
def _sc_gather_kernel():
    @functools.partial(
        pl.kernel,
        out_type=jax.ShapeDtypeStruct((ES, HDIM), jnp.float32),
        mesh=plsc.VectorSubcoreMesh(**_MESH),
        scratch_types=[
            pltpu.VMEM((NCHUNK, C), jnp.int32),
            pltpu.VMEM((C, HDIM), jnp.float32),
            pltpu.VMEM((C, HDIM), jnp.float32),
            pltpu.SemaphoreType.DMA,
            pltpu.SemaphoreType.DMA,
        ],
    )
    def _sc_gather(h_hbm, src_hbm, out_hbm, idx_v, buf0, buf1, sem0, sem1):
        wid = lax.axis_index("s") * NC + lax.axis_index("c")
        pltpu.sync_copy(src_hbm.at[wid], idx_v)
        bufs = (buf0, buf1)
        sems = (sem0, sem1)

        pltpu.async_copy(h_hbm.at[idx_v.at[0]], buf0, sem0)

        def body(ci, carry):
            for par in range(2):
                @pl.when(ci % 2 == par)
                def _():
                    nxt = 1 - par

                    @pl.when(ci + 1 < NCHUNK)
                    def _():
                        pltpu.async_copy(h_hbm.at[idx_v.at[ci + 1]],
                                         bufs[nxt], sems[nxt])

                    pltpu.make_async_copy(h_hbm.at[pl.ds(0, C)],
                                          bufs[par], sems[par]).wait()
                    pltpu.sync_copy(bufs[par],
                                    out_hbm.at[pl.ds(wid * EW + ci * C, C)])
            return carry

        lax.fori_loop(0, NCHUNK, body, 0)

    return _sc_gather


@functools.cache
def _sc_scatter_kernel():
    @functools.partial(
        pl.kernel,
        out_type=jax.ShapeDtypeStruct((NC, NPAD, HDIM), jnp.float32),
        mesh=plsc.VectorSubcoreMesh(**_MESH),
        scratch_types=[
            pltpu.VMEM((NCHUNK, C), jnp.int32),
            pltpu.VMEM((C, HDIM), jnp.float32),
            pltpu.VMEM((C, HDIM), jnp.float32),
            pltpu.VMEM_SHARED((NPAD, HDIM), jnp.float32),
            pltpu.SemaphoreType.DMA,
            pltpu.SemaphoreType.DMA,
        ],
    )
    def _sc_scatter(msg_hbm, dst_hbm, zeros_hbm, out_hbm,
                    idx_v, buf0, buf1, acc_sh, sem0, sem1):
        cid = lax.axis_index("c")
        sid = lax.axis_index("s")
        wid = sid * NC + cid
        pltpu.sync_copy(zeros_hbm, acc_sh.at[pl.ds(sid * NSTR, NSTR)])
        plsc.subcore_barrier()
        pltpu.sync_copy(dst_hbm.at[wid], idx_v)
        bufs = (buf0, buf1)
        sems = (sem0, sem1)

        pltpu.async_copy(msg_hbm.at[pl.ds(wid * EW, C)], buf0, sem0)

        def body(ci, carry):
            for par in range(2):
                @pl.when(ci % 2 == par)
                def _():
                    nxt = 1 - par

                    @pl.when(ci + 1 < NCHUNK)
                    def _():
                        pltpu.async_copy(
                            msg_hbm.at[pl.ds(wid * EW + (ci + 1) * C, C)],
                            bufs[nxt], sems[nxt])

                    pltpu.make_async_copy(msg_hbm.at[pl.ds(0, C)],
                                          bufs[par], sems[par]).wait()
                    pltpu.sync_copy(bufs[par], acc_sh.at[idx_v.at[ci]],
                                    add=True)
            return carry

        lax.fori_loop(0, NCHUNK, body, 0)
        plsc.subcore_barrier()
        pltpu.sync_copy(acc_sh.at[pl.ds(sid * NSTR, NSTR)],
                        out_hbm.at[cid, pl.ds(sid * NSTR, NSTR)])

    return _sc_scatter


def _edge_body(ea_ref, hs_ref, wi_ref, whh_ref, bi_ref, bh_ref, out_ref):
    gi = jnp.dot(ea_ref[...], wi_ref[...],
                 preferred_element_type=jnp.float32) + bi_ref[...]
    gh = jnp.dot(hs_ref[...], whh_ref[...],
                 preferred_element_type=jnp.float32) + bh_ref[...]
    r = jax.nn.sigmoid(gi[:, :HDIM] + gh[:, :HDIM])
    z = jax.nn.sigmoid(gi[:, HDIM:2 * HDIM] + gh[:, HDIM:2 * HDIM])
    n = jnp.tanh(gi[:, 2 * HDIM:] + r * gh[:, 2 * HDIM:])
    out_ref[...] = (1.0 - z) * n + z * hs_ref[...]


BE = 2000
GE = ES // BE


def _tc_edge(edge_attr, h_src, wi, whh, bi, bh):
    return pl.pallas_call(
        _edge_body,
        grid=(GE,),
        in_specs=[
            pl.BlockSpec((BE, RELDIM), lambda i: (i, 0)),
            pl.BlockSpec((BE, HDIM), lambda i: (i, 0)),
            pl.BlockSpec((RELDIM, 3 * HDIM), lambda i: (0, 0)),
            pl.BlockSpec((HDIM, 3 * HDIM), lambda i: (0, 0)),
            pl.BlockSpec((1, 3 * HDIM), lambda i: (0, 0)),
            pl.BlockSpec((1, 3 * HDIM), lambda i: (0, 0)),
        ],
        out_specs=pl.BlockSpec((BE, HDIM), lambda i: (i, 0)),
        out_shape=jax.ShapeDtypeStruct((ES, HDIM), jnp.float32),
    )(edge_attr, h_src, wi, whh, bi, bh)


def _node_body(x_ref, *rest):
    parts = rest[:NSLICE]
    wi_ref, whh_ref, bi_ref, bh_ref, out_ref = rest[NSLICE:]
    red = parts[0][0] + parts[0][1]
    for p in parts[1:]:
        red = red + p[0] + p[1]
    gi = jnp.dot(x_ref[...], wi_ref[...],
                 preferred_element_type=jnp.float32) + bi_ref[...]
    gh = jnp.dot(red, whh_ref[...],
                 preferred_element_type=jnp.float32) + bh_ref[...]
    r = jax.nn.sigmoid(gi[:, :HDIM] + gh[:, :HDIM])
    z = jax.nn.sigmoid(gi[:, HDIM:2 * HDIM] + gh[:, HDIM:2 * HDIM])
    n = jnp.tanh(gi[:, 2 * HDIM:] + r * gh[:, 2 * HDIM:])
    out_ref[...] = (1.0 - z) * n + z * red


BN = 1000
GN = N // BN


def _tc_node(x, parts_list, wi, whh, bi, bh):
    part_spec = pl.BlockSpec((NC, BN, HDIM), lambda i: (0, i, 0))
    return pl.pallas_call(
        _node_body,
        grid=(GN,),
        in_specs=[pl.BlockSpec((BN, NODEDIM), lambda i: (i, 0))]
        + [part_spec] * NSLICE
        + [
            pl.BlockSpec((NODEDIM, 3 * HDIM), lambda i: (0, 0)),
            pl.BlockSpec((HDIM, 3 * HDIM), lambda i: (0, 0)),
            pl.BlockSpec((1, 3 * HDIM), lambda i: (0, 0)),
            pl.BlockSpec((1, 3 * HDIM), lambda i: (0, 0)),
        ],
        out_specs=pl.BlockSpec((BN, HDIM), lambda i: (i, 0)),
        out_shape=jax.ShapeDtypeStruct((N, HDIM), jnp.float32),
    )(x, *parts_list, wi, whh, bi, bh)


def kernel(x, h, edge_index, edge_attr, W_ih_rel, W_hh_rel, b_ih_rel,
           b_hh_rel, W_ih_node, W_hh_node, b_ih_node, b_hh_node):
    src = edge_index[0].reshape(NSLICE, NW, NCHUNK, C)
    dst = edge_index[1].reshape(NSLICE, NW, NCHUNK, C)
    zeros = jnp.zeros((NSTR, HDIM), jnp.float32)
    wi_rel = W_ih_rel.T
    whh_rel = W_hh_rel.T
    bi_rel = b_ih_rel.reshape(1, -1)
    bh_rel = b_hh_rel.reshape(1, -1)

    parts_list = []
    for s in range(NSLICE):
        h_src = _sc_gather_kernel()(h, src[s])
        msg = _tc_edge(edge_attr[s * ES:(s + 1) * ES], h_src,
                       wi_rel, whh_rel, bi_rel, bh_rel)
        parts_list.append(_sc_scatter_kernel()(msg, dst[s], zeros))

    return _tc_node(x, parts_list,
                    W_ih_node.T, W_hh_node.T,
                    b_ih_node.reshape(1, -1), b_hh_node.reshape(1, -1))


# speedup vs baseline: 1.7897x; 1.0190x over previous
import functools

import jax
import jax.numpy as jnp
from jax import lax
from jax.experimental import pallas as pl
from jax.experimental.pallas import tpu as pltpu
from jax.experimental.pallas import tpu_sc as plsc

N = 10000
E = 320000
NODEDIM = 128
RELDIM = 16
HDIM = 128

NC = 2
NS = 16
NW = NC * NS
NSLICE = 5
ES = E // NSLICE
EW = ES // NW
C = 80
NCHUNK = EW // C
NPAD = 10240
NSTR = NPAD // NS

_MESH = dict(core_axis_name="c", subcore_axis_name="s",
             num_cores=NC, num_subcores=NS)


@functools.cache
def _sc_gather_kernel():
    @functools.partial(
        pl.kernel,
        out_type=jax.ShapeDtypeStruct((ES, HDIM), jnp.float32),
        mesh=plsc.VectorSubcoreMesh(**_MESH),
        scratch_types=[
            pltpu.VMEM((NCHUNK, C), jnp.int32),
            pltpu.VMEM((C, HDIM), jnp.float32),
            pltpu.VMEM((C, HDIM), jnp.float32),
            pltpu.SemaphoreType.DMA,
            pltpu.SemaphoreType.DMA,
            pltpu.SemaphoreType.DMA,
            pltpu.SemaphoreType.DMA,
        ],
    )
    def _sc_gather(h_hbm, src_hbm, out_hbm, idx_v,
                   buf0, buf1, rsem0, rsem1, wsem0, wsem1):
        wid = lax.axis_index("s") * NC + lax.axis_index("c")
        pltpu.sync_copy(src_hbm.at[wid], idx_v)
        bufs = (buf0, buf1)
        rsems = (rsem0, rsem1)
        wsems = (wsem0, wsem1)

        pltpu.async_copy(h_hbm.at[idx_v.at[0]], buf0, rsem0)

        def body(ci, carry):
            for par in range(2):
                @pl.when(ci % 2 == par)
                def _():
                    nxt = 1 - par

                    @pl.when(ci + 1 < NCHUNK)
                    def _():
                        @pl.when(ci >= 1)
                        def _():
                            pltpu.make_async_copy(
                                bufs[nxt], out_hbm.at[pl.ds(0, C)],
                                wsems[nxt]).wait()

                        pltpu.async_copy(h_hbm.at[idx_v.at[ci + 1]],
                                         bufs[nxt], rsems[nxt])

                    pltpu.make_async_copy(h_hbm.at[pl.ds(0, C)],
                                          bufs[par], rsems[par]).wait()
                    pltpu.async_copy(bufs[par],
                                     out_hbm.at[pl.ds(wid * EW + ci * C, C)],
                                     wsems[par])
            return carry

        lax.fori_loop(0, NCHUNK, body, 0)
        for par in range(2):
            pltpu.make_async_copy(bufs[par], out_hbm.at[pl.ds(0, C)],
                                  wsems[par]).wait()

    return _sc_gather


@functools.cache
def _sc_scatter_kernel():
    @functools.partial(
        pl.kernel,
        out_type=jax.ShapeDtypeStruct((NC, NPAD, HDIM), jnp.float32),
        mesh=plsc.VectorSubcoreMesh(**_MESH),
        scratch_types=[
            pltpu.VMEM((NCHUNK, C), jnp.int32),
            pltpu.VMEM((C, HDIM), jnp.float32),
            pltpu.VMEM((C, HDIM), jnp.float32),
            pltpu.VMEM((C, HDIM), jnp.float32),
            pltpu.VMEM_SHARED((NPAD, HDIM), jnp.float32),
            pltpu.SemaphoreType.DMA,
            pltpu.SemaphoreType.DMA,
            pltpu.SemaphoreType.DMA,
            pltpu.SemaphoreType.DMA,
        ],
    )
    def _sc_scatter(msg_hbm, dst_hbm, out_hbm,
                    idx_v, buf0, buf1, zbuf, acc_sh,
                    rsem0, rsem1, asem0, asem1):
        cid = lax.axis_index("c")
        sid = lax.axis_index("s")
        wid = sid * NC + cid
        bufs = (buf0, buf1)
        rsems = (rsem0, rsem1)
        asems = (asem0, asem1)

        pltpu.async_copy(msg_hbm.at[pl.ds(wid * EW, C)], buf0, rsem0)
        pltpu.sync_copy(dst_hbm.at[wid], idx_v)

        zero16 = jnp.zeros((16,), jnp.float32)

        def zrow(i, carry):
            for j in range(HDIM // 16):
                zbuf[i, pl.ds(j * 16, 16)] = zero16
            return carry

        lax.fori_loop(0, C, zrow, 0)
        for rep in range(NSTR // C):
            pltpu.sync_copy(zbuf, acc_sh.at[pl.ds(sid * NSTR + rep * C, C)])
        plsc.subcore_barrier()

        def body(ci, carry):
            for par in range(2):
                @pl.when(ci % 2 == par)
                def _():
                    nxt = 1 - par

                    @pl.when(ci + 1 < NCHUNK)
                    def _():
                        @pl.when(ci >= 1)
                        def _():
                            pltpu.make_async_copy(
                                bufs[nxt], acc_sh.at[pl.ds(0, C)],
                                asems[nxt]).wait()

                        pltpu.async_copy(
                            msg_hbm.at[pl.ds(wid * EW + (ci + 1) * C, C)],
                            bufs[nxt], rsems[nxt])

                    pltpu.make_async_copy(msg_hbm.at[pl.ds(0, C)],
                                          bufs[par], rsems[par]).wait()
                    pltpu.async_copy(bufs[par], acc_sh.at[idx_v.at[ci]],
                                     asems[par], add=True)
            return carry

        lax.fori_loop(0, NCHUNK, body, 0)
        for par in range(2):
            pltpu.make_async_copy(bufs[par], acc_sh.at[pl.ds(0, C)],
                                  asems[par]).wait()
        plsc.subcore_barrier()
        pltpu.sync_copy(acc_sh.at[pl.ds(sid * NSTR, NSTR)],
                        out_hbm.at[cid, pl.ds(sid * NSTR, NSTR)])

    return _sc_scatter


def _edge_body(ea_ref, hs_ref, wi_ref, whh_ref, bi_ref, bh_ref, out_ref):
    gi = jnp.dot(ea_ref[...], wi_ref[...],
                 preferred_element_type=jnp.float32) + bi_ref[...]
    gh = jnp.dot(hs_ref[...], whh_ref[...],
                 preferred_element_type=jnp.float32) + bh_ref[...]
    r = jax.nn.sigmoid(gi[:, :HDIM] + gh[:, :HDIM])
    z = jax.nn.sigmoid(gi[:, HDIM:2 * HDIM] + gh[:, HDIM:2 * HDIM])
    n = jnp.tanh(gi[:, 2 * HDIM:] + r * gh[:, 2 * HDIM:])
    out_ref[...] = (1.0 - z) * n + z * hs_ref[...]


BE = 2000
GE = ES // BE


def _tc_edge(edge_attr, h_src, wi, whh, bi, bh):
    return pl.pallas_call(
        _edge_body,
        grid=(GE,),
        in_specs=[
            pl.BlockSpec((BE, RELDIM), lambda i: (i, 0)),
            pl.BlockSpec((BE, HDIM), lambda i: (i, 0)),
            pl.BlockSpec((RELDIM, 3 * HDIM), lambda i: (0, 0)),
            pl.BlockSpec((HDIM, 3 * HDIM), lambda i: (0, 0)),
            pl.BlockSpec((1, 3 * HDIM), lambda i: (0, 0)),
            pl.BlockSpec((1, 3 * HDIM), lambda i: (0, 0)),
        ],
        out_specs=pl.BlockSpec((BE, HDIM), lambda i: (i, 0)),
        out_shape=jax.ShapeDtypeStruct((ES, HDIM), jnp.float32),
    )(edge_attr, h_src, wi, whh, bi, bh)


def _node_body(x_ref, *rest):
    parts = rest[:NSLICE]
    wi_ref, whh_ref, bi_ref, bh_ref, out_ref = rest[NSLICE:]
    red = parts[0][0] + parts[0][1]
    for p in parts[1:]:
        red = red + p[0] + p[1]
    gi = jnp.dot(x_ref[...], wi_ref[...],
                 preferred_element_type=jnp.float32) + bi_ref[...]
    gh = jnp.dot(red, whh_ref[...],
                 preferred_element_type=jnp.float32) + bh_ref[...]
    r = jax.nn.sigmoid(gi[:, :HDIM] + gh[:, :HDIM])
    z = jax.nn.sigmoid(gi[:, HDIM:2 * HDIM] + gh[:, HDIM:2 * HDIM])
    n = jnp.tanh(gi[:, 2 * HDIM:] + r * gh[:, 2 * HDIM:])
    out_ref[...] = (1.0 - z) * n + z * red


BN = 1000
GN = N // BN


def _tc_node(x, parts_list, wi, whh, bi, bh):
    part_spec = pl.BlockSpec((NC, BN, HDIM), lambda i: (0, i, 0))
    return pl.pallas_call(
        _node_body,
        grid=(GN,),
        in_specs=[pl.BlockSpec((BN, NODEDIM), lambda i: (i, 0))]
        + [part_spec] * NSLICE
        + [
            pl.BlockSpec((NODEDIM, 3 * HDIM), lambda i: (0, 0)),
            pl.BlockSpec((HDIM, 3 * HDIM), lambda i: (0, 0)),
            pl.BlockSpec((1, 3 * HDIM), lambda i: (0, 0)),
            pl.BlockSpec((1, 3 * HDIM), lambda i: (0, 0)),
        ],
        out_specs=pl.BlockSpec((BN, HDIM), lambda i: (i, 0)),
        out_shape=jax.ShapeDtypeStruct((N, HDIM), jnp.float32),
    )(x, *parts_list, wi, whh, bi, bh)


def kernel(x, h, edge_index, edge_attr, W_ih_rel, W_hh_rel, b_ih_rel,
           b_hh_rel, W_ih_node, W_hh_node, b_ih_node, b_hh_node):
    src = edge_index[0].reshape(NSLICE, NW, NCHUNK, C)
    dst = edge_index[1].reshape(NSLICE, NW, NCHUNK, C)
    wi_rel = W_ih_rel.T
    whh_rel = W_hh_rel.T
    bi_rel = b_ih_rel.reshape(1, -1)
    bh_rel = b_hh_rel.reshape(1, -1)

    parts_list = []
    for s in range(NSLICE):
        h_src = _sc_gather_kernel()(h, src[s])
        msg = _tc_edge(edge_attr[s * ES:(s + 1) * ES], h_src,
                       wi_rel, whh_rel, bi_rel, bh_rel)
        parts_list.append(_sc_scatter_kernel()(msg, dst[s]))

    return _tc_node(x, parts_list,
                    W_ih_node.T, W_hh_node.T,
                    b_ih_node.reshape(1, -1), b_hh_node.reshape(1, -1))


# speedup vs baseline: 1.8583x; 1.0383x over previous
import functools

import jax
import jax.numpy as jnp
from jax import lax
from jax.experimental import pallas as pl
from jax.experimental.pallas import tpu as pltpu
from jax.experimental.pallas import tpu_sc as plsc

N = 10000
E = 320000
NODEDIM = 128
RELDIM = 16
HDIM = 128

NC = 2
NS = 16
NW = NC * NS
NSLICE = 5
ES = E // NSLICE
EW = ES // NW
C = 80
NCHUNK = EW // C
NPAD = 10240
NSTR = NPAD // NS

_MESH = dict(core_axis_name="c", subcore_axis_name="s",
             num_cores=NC, num_subcores=NS)


@functools.cache
def _sc_gather_kernel(s):
    @functools.partial(
        pl.kernel,
        out_type=jax.ShapeDtypeStruct((ES, HDIM), jnp.float32),
        mesh=plsc.VectorSubcoreMesh(**_MESH),
        scratch_types=[
            pltpu.VMEM((NCHUNK, C), jnp.int32),
            pltpu.VMEM((C, HDIM), jnp.float32),
            pltpu.VMEM((C, HDIM), jnp.float32),
            pltpu.SemaphoreType.DMA,
            pltpu.SemaphoreType.DMA,
            pltpu.SemaphoreType.DMA,
            pltpu.SemaphoreType.DMA,
        ],
    )
    def _sc_gather(h_hbm, src_hbm, out_hbm, idx_v,
                   buf0, buf1, rsem0, rsem1, wsem0, wsem1):
        wid = lax.axis_index("s") * NC + lax.axis_index("c")
        pltpu.sync_copy(src_hbm.at[s, wid], idx_v)
        bufs = (buf0, buf1)
        rsems = (rsem0, rsem1)
        wsems = (wsem0, wsem1)

        pltpu.async_copy(h_hbm.at[idx_v.at[0]], buf0, rsem0)

        def body(ci, carry):
            for par in range(2):
                @pl.when(ci % 2 == par)
                def _():
                    nxt = 1 - par

                    @pl.when(ci + 1 < NCHUNK)
                    def _():
                        @pl.when(ci >= 1)
                        def _():
                            pltpu.make_async_copy(
                                bufs[nxt], out_hbm.at[pl.ds(0, C)],
                                wsems[nxt]).wait()

                        pltpu.async_copy(h_hbm.at[idx_v.at[ci + 1]],
                                         bufs[nxt], rsems[nxt])

                    pltpu.make_async_copy(h_hbm.at[pl.ds(0, C)],
                                          bufs[par], rsems[par]).wait()
                    pltpu.async_copy(bufs[par],
                                     out_hbm.at[pl.ds(wid * EW + ci * C, C)],
                                     wsems[par])
            return carry

        lax.fori_loop(0, NCHUNK, body, 0)
        for par in range(2):
            pltpu.make_async_copy(bufs[par], out_hbm.at[pl.ds(0, C)],
                                  wsems[par]).wait()

    return _sc_gather


@functools.cache
def _sc_scatter_kernel(s):
    @functools.partial(
        pl.kernel,
        out_type=jax.ShapeDtypeStruct((NC, NPAD, HDIM), jnp.float32),
        mesh=plsc.VectorSubcoreMesh(**_MESH),
        scratch_types=[
            pltpu.VMEM((NCHUNK, C), jnp.int32),
            pltpu.VMEM((C, HDIM), jnp.float32),
            pltpu.VMEM((C, HDIM), jnp.float32),
            pltpu.VMEM((C, HDIM), jnp.float32),
            pltpu.VMEM_SHARED((NPAD, HDIM), jnp.float32),
            pltpu.SemaphoreType.DMA,
            pltpu.SemaphoreType.DMA,
            pltpu.SemaphoreType.DMA,
            pltpu.SemaphoreType.DMA,
        ],
    )
    def _sc_scatter(msg_hbm, dst_hbm, out_hbm,
                    idx_v, buf0, buf1, zbuf, acc_sh,
                    rsem0, rsem1, asem0, asem1):
        cid = lax.axis_index("c")
        sid = lax.axis_index("s")
        wid = sid * NC + cid
        bufs = (buf0, buf1)
        rsems = (rsem0, rsem1)
        asems = (asem0, asem1)

        pltpu.async_copy(msg_hbm.at[pl.ds(wid * EW, C)], buf0, rsem0)
        pltpu.sync_copy(dst_hbm.at[s, wid], idx_v)

        zero16 = jnp.zeros((16,), jnp.float32)

        def zrow(i, carry):
            for j in range(HDIM // 16):
                zbuf[i, pl.ds(j * 16, 16)] = zero16
            return carry

        lax.fori_loop(0, C, zrow, 0)
        for rep in range(NSTR // C):
            pltpu.sync_copy(zbuf, acc_sh.at[pl.ds(sid * NSTR + rep * C, C)])
        plsc.subcore_barrier()

        def body(ci, carry):
            for par in range(2):
                @pl.when(ci % 2 == par)
                def _():
                    nxt = 1 - par

                    @pl.when(ci + 1 < NCHUNK)
                    def _():
                        @pl.when(ci >= 1)
                        def _():
                            pltpu.make_async_copy(
                                bufs[nxt], acc_sh.at[pl.ds(0, C)],
                                asems[nxt]).wait()

                        pltpu.async_copy(
                            msg_hbm.at[pl.ds(wid * EW + (ci + 1) * C, C)],
                            bufs[nxt], rsems[nxt])

                    pltpu.make_async_copy(msg_hbm.at[pl.ds(0, C)],
                                          bufs[par], rsems[par]).wait()
                    pltpu.async_copy(bufs[par], acc_sh.at[idx_v.at[ci]],
                                     asems[par], add=True)
            return carry

        lax.fori_loop(0, NCHUNK, body, 0)
        for par in range(2):
            pltpu.make_async_copy(bufs[par], acc_sh.at[pl.ds(0, C)],
                                  asems[par]).wait()
        plsc.subcore_barrier()
        pltpu.sync_copy(acc_sh.at[pl.ds(sid * NSTR, NSTR)],
                        out_hbm.at[cid, pl.ds(sid * NSTR, NSTR)])

    return _sc_scatter


def _edge_body(ea_ref, hs_ref, wi_ref, whh_ref, bi_ref, bh_ref, out_ref):
    gi = jnp.dot(ea_ref[...], wi_ref[...],
                 preferred_element_type=jnp.float32) + bi_ref[...]
    gh = jnp.dot(hs_ref[...], whh_ref[...],
                 preferred_element_type=jnp.float32) + bh_ref[...]
    r = jax.nn.sigmoid(gi[:, :HDIM] + gh[:, :HDIM])
    z = jax.nn.sigmoid(gi[:, HDIM:2 * HDIM] + gh[:, HDIM:2 * HDIM])
    n = jnp.tanh(gi[:, 2 * HDIM:] + r * gh[:, 2 * HDIM:])
    out_ref[...] = (1.0 - z) * n + z * hs_ref[...]


BE = 2000
GE = ES // BE


def _tc_edge(s, edge_attr, h_src, wi, whh, bi, bh):
    return pl.pallas_call(
        _edge_body,
        grid=(GE,),
        in_specs=[
            pl.BlockSpec((BE, RELDIM), lambda i, s=s: (i + s * GE, 0)),
            pl.BlockSpec((BE, HDIM), lambda i: (i, 0)),
            pl.BlockSpec((RELDIM, 3 * HDIM), lambda i: (0, 0)),
            pl.BlockSpec((HDIM, 3 * HDIM), lambda i: (0, 0)),
            pl.BlockSpec((1, 3 * HDIM), lambda i: (0, 0)),
            pl.BlockSpec((1, 3 * HDIM), lambda i: (0, 0)),
        ],
        out_specs=pl.BlockSpec((BE, HDIM), lambda i: (i, 0)),
        out_shape=jax.ShapeDtypeStruct((ES, HDIM), jnp.float32),
    )(edge_attr, h_src, wi, whh, bi, bh)


def _node_body(x_ref, *rest):
    parts = rest[:NSLICE]
    wi_ref, whh_ref, bi_ref, bh_ref, out_ref = rest[NSLICE:]
    red = parts[0][0] + parts[0][1]
    for p in parts[1:]:
        red = red + p[0] + p[1]
    gi = jnp.dot(x_ref[...], wi_ref[...],
                 preferred_element_type=jnp.float32) + bi_ref[...]
    gh = jnp.dot(red, whh_ref[...],
                 preferred_element_type=jnp.float32) + bh_ref[...]
    r = jax.nn.sigmoid(gi[:, :HDIM] + gh[:, :HDIM])
    z = jax.nn.sigmoid(gi[:, HDIM:2 * HDIM] + gh[:, HDIM:2 * HDIM])
    n = jnp.tanh(gi[:, 2 * HDIM:] + r * gh[:, 2 * HDIM:])
    out_ref[...] = (1.0 - z) * n + z * red


BN = 1000
GN = N // BN


def _tc_node(x, parts_list, wi, whh, bi, bh):
    part_spec = pl.BlockSpec((NC, BN, HDIM), lambda i: (0, i, 0))
    return pl.pallas_call(
        _node_body,
        grid=(GN,),
        in_specs=[pl.BlockSpec((BN, NODEDIM), lambda i: (i, 0))]
        + [part_spec] * NSLICE
        + [
            pl.BlockSpec((NODEDIM, 3 * HDIM), lambda i: (0, 0)),
            pl.BlockSpec((HDIM, 3 * HDIM), lambda i: (0, 0)),
            pl.BlockSpec((1, 3 * HDIM), lambda i: (0, 0)),
            pl.BlockSpec((1, 3 * HDIM), lambda i: (0, 0)),
        ],
        out_specs=pl.BlockSpec((BN, HDIM), lambda i: (i, 0)),
        out_shape=jax.ShapeDtypeStruct((N, HDIM), jnp.float32),
    )(x, *parts_list, wi, whh, bi, bh)


def kernel(x, h, edge_index, edge_attr, W_ih_rel, W_hh_rel, b_ih_rel,
           b_hh_rel, W_ih_node, W_hh_node, b_ih_node, b_hh_node):
    src = edge_index[0].reshape(NSLICE, NW, NCHUNK, C)
    dst = edge_index[1].reshape(NSLICE, NW, NCHUNK, C)
    wi_rel = W_ih_rel.T
    whh_rel = W_hh_rel.T
    bi_rel = b_ih_rel.reshape(1, -1)
    bh_rel = b_hh_rel.reshape(1, -1)

    parts_list = []
    for s in range(NSLICE):
        h_src = _sc_gather_kernel(s)(h, src)
        msg = _tc_edge(s, edge_attr, h_src,
                       wi_rel, whh_rel, bi_rel, bh_rel)
        parts_list.append(_sc_scatter_kernel(s)(msg, dst))

    return _tc_node(x, parts_list,
                    W_ih_node.T, W_hh_node.T,
                    b_ih_node.reshape(1, -1), b_hh_node.reshape(1, -1))


# speedup vs baseline: 2.3201x; 1.2485x over previous
import functools

import jax
import jax.numpy as jnp
from jax import lax
from jax.experimental import pallas as pl
from jax.experimental.pallas import tpu as pltpu
from jax.experimental.pallas import tpu_sc as plsc

N = 10000
E = 320000
NODEDIM = 128
RELDIM = 16
HDIM = 128

NC = 2
NS = 16
NW = NC * NS
NSLICE = 5
ES = E // NSLICE
EW = ES // NW
C = 80
NCHUNK = EW // C
NPAD = 10240
NSTR = NPAD // NS

_MESH = dict(core_axis_name="c", subcore_axis_name="s",
             num_cores=NC, num_subcores=NS)


@functools.cache
def _sc_gather_kernel(s):
    @functools.partial(
        pl.kernel,
        out_type=jax.ShapeDtypeStruct((ES, HDIM), jnp.float32),
        mesh=plsc.VectorSubcoreMesh(**_MESH),
        scratch_types=[
            pltpu.VMEM((NCHUNK, C), jnp.int32),
            pltpu.VMEM((C, HDIM), jnp.float32),
            pltpu.VMEM((C, HDIM), jnp.float32),
            pltpu.SemaphoreType.DMA,
            pltpu.SemaphoreType.DMA,
            pltpu.SemaphoreType.DMA,
            pltpu.SemaphoreType.DMA,
        ],
    )
    def _sc_gather(h_hbm, src_hbm, out_hbm, idx_v,
                   buf0, buf1, rsem0, rsem1, wsem0, wsem1):
        wid = lax.axis_index("s") * NC + lax.axis_index("c")
        pltpu.sync_copy(src_hbm.at[s, wid], idx_v)
        bufs = (buf0, buf1)
        rsems = (rsem0, rsem1)
        wsems = (wsem0, wsem1)

        pltpu.async_copy(h_hbm.at[idx_v.at[0]], buf0, rsem0)

        def body(ci, carry):
            for par in range(2):
                @pl.when(ci % 2 == par)
                def _():
                    nxt = 1 - par

                    @pl.when(ci + 1 < NCHUNK)
                    def _():
                        @pl.when(ci >= 1)
                        def _():
                            pltpu.make_async_copy(
                                bufs[nxt], out_hbm.at[pl.ds(0, C)],
                                wsems[nxt]).wait()

                        pltpu.async_copy(h_hbm.at[idx_v.at[ci + 1]],
                                         bufs[nxt], rsems[nxt])

                    pltpu.make_async_copy(h_hbm.at[pl.ds(0, C)],
                                          bufs[par], rsems[par]).wait()
                    pltpu.async_copy(bufs[par],
                                     out_hbm.at[pl.ds(wid * EW + ci * C, C)],
                                     wsems[par])
            return carry

        lax.fori_loop(0, NCHUNK, body, 0)
        for par in range(2):
            pltpu.make_async_copy(bufs[par], out_hbm.at[pl.ds(0, C)],
                                  wsems[par]).wait()

    return _sc_gather


@functools.cache
def _sc_scatter_kernel(s):
    @functools.partial(
        pl.kernel,
        out_type=jax.ShapeDtypeStruct((NC, NPAD, HDIM), jnp.float32),
        mesh=plsc.VectorSubcoreMesh(**_MESH),
        scratch_types=[
            pltpu.VMEM((NCHUNK, C), jnp.int32),
            pltpu.VMEM((C, HDIM), jnp.float32),
            pltpu.VMEM((C, HDIM), jnp.float32),
            pltpu.VMEM((C, HDIM), jnp.float32),
            pltpu.VMEM_SHARED((NPAD, HDIM), jnp.float32),
            pltpu.SemaphoreType.DMA,
            pltpu.SemaphoreType.DMA,
            pltpu.SemaphoreType.DMA,
            pltpu.SemaphoreType.DMA,
        ],
    )
    def _sc_scatter(msg_hbm, dst_hbm, out_hbm,
                    idx_v, buf0, buf1, zbuf, acc_sh,
                    rsem0, rsem1, asem0, asem1):
        cid = lax.axis_index("c")
        sid = lax.axis_index("s")
        wid = sid * NC + cid
        bufs = (buf0, buf1)
        rsems = (rsem0, rsem1)
        asems = (asem0, asem1)

        pltpu.async_copy(msg_hbm.at[pl.ds(wid * EW, C)], buf0, rsem0)
        pltpu.sync_copy(dst_hbm.at[s, wid], idx_v)

        zero16 = jnp.zeros((16,), jnp.float32)

        def zrow(i, carry):
            for j in range(HDIM // 16):
                zbuf[i, pl.ds(j * 16, 16)] = zero16
            return carry

        lax.fori_loop(0, C, zrow, 0)
        for rep in range(NSTR // C):
            pltpu.sync_copy(zbuf, acc_sh.at[pl.ds(sid * NSTR + rep * C, C)])
        plsc.subcore_barrier()

        def body(ci, carry):
            for par in range(2):
                @pl.when(ci % 2 == par)
                def _():
                    nxt = 1 - par

                    @pl.when(ci + 1 < NCHUNK)
                    def _():
                        @pl.when(ci >= 1)
                        def _():
                            pltpu.make_async_copy(
                                bufs[nxt], acc_sh.at[pl.ds(0, C)],
                                asems[nxt]).wait()

                        pltpu.async_copy(
                            msg_hbm.at[pl.ds(wid * EW + (ci + 1) * C, C)],
                            bufs[nxt], rsems[nxt])

                    pltpu.make_async_copy(msg_hbm.at[pl.ds(0, C)],
                                          bufs[par], rsems[par]).wait()
                    pltpu.async_copy(bufs[par], acc_sh.at[idx_v.at[ci]],
                                     asems[par], add=True)
            return carry

        lax.fori_loop(0, NCHUNK, body, 0)
        for par in range(2):
            pltpu.make_async_copy(bufs[par], acc_sh.at[pl.ds(0, C)],
                                  asems[par]).wait()
        plsc.subcore_barrier()
        pltpu.sync_copy(acc_sh.at[pl.ds(sid * NSTR, NSTR)],
                        out_hbm.at[cid, pl.ds(sid * NSTR, NSTR)])

    return _sc_scatter


def _edge_body(ea_ref, hs_ref, wi_ref, whh_ref, bi_ref, bh_ref, out_ref):
    gi = lax.dot_general(ea_ref[...], wi_ref[...],
                         (((0,), (0,)), ((), ())),
                         preferred_element_type=jnp.float32) + bi_ref[...]
    gh = jnp.dot(hs_ref[...], whh_ref[...],
                 preferred_element_type=jnp.float32) + bh_ref[...]
    r = jax.nn.sigmoid(gi[:, :HDIM] + gh[:, :HDIM])
    z = jax.nn.sigmoid(gi[:, HDIM:2 * HDIM] + gh[:, HDIM:2 * HDIM])
    n = jnp.tanh(gi[:, 2 * HDIM:] + r * gh[:, 2 * HDIM:])
    out_ref[...] = (1.0 - z) * n + z * hs_ref[...]


BE = 2560
GE = ES // BE


def _tc_edge(s, edge_attr, h_src, wi, whh, bi, bh):
    return pl.pallas_call(
        _edge_body,
        grid=(GE,),
        in_specs=[
            pl.BlockSpec((RELDIM, BE), lambda i, s=s: (0, i + s * GE)),
            pl.BlockSpec((BE, HDIM), lambda i: (i, 0)),
            pl.BlockSpec((RELDIM, 3 * HDIM), lambda i: (0, 0)),
            pl.BlockSpec((HDIM, 3 * HDIM), lambda i: (0, 0)),
            pl.BlockSpec((1, 3 * HDIM), lambda i: (0, 0)),
            pl.BlockSpec((1, 3 * HDIM), lambda i: (0, 0)),
        ],
        out_specs=pl.BlockSpec((BE, HDIM), lambda i: (i, 0)),
        out_shape=jax.ShapeDtypeStruct((ES, HDIM), jnp.float32),
    )(edge_attr, h_src, wi, whh, bi, bh)


def _node_body(x_ref, *rest):
    parts = rest[:NSLICE]
    wi_ref, whh_ref, bi_ref, bh_ref, out_ref = rest[NSLICE:]
    red = parts[0][0] + parts[0][1]
    for p in parts[1:]:
        red = red + p[0] + p[1]
    gi = jnp.dot(x_ref[...], wi_ref[...],
                 preferred_element_type=jnp.float32) + bi_ref[...]
    gh = jnp.dot(red, whh_ref[...],
                 preferred_element_type=jnp.float32) + bh_ref[...]
    r = jax.nn.sigmoid(gi[:, :HDIM] + gh[:, :HDIM])
    z = jax.nn.sigmoid(gi[:, HDIM:2 * HDIM] + gh[:, HDIM:2 * HDIM])
    n = jnp.tanh(gi[:, 2 * HDIM:] + r * gh[:, 2 * HDIM:])
    out_ref[...] = (1.0 - z) * n + z * red


BN = 1000
GN = N // BN


def _tc_node(x, parts_list, wi, whh, bi, bh):
    part_spec = pl.BlockSpec((NC, BN, HDIM), lambda i: (0, i, 0))
    return pl.pallas_call(
        _node_body,
        grid=(GN,),
        in_specs=[pl.BlockSpec((BN, NODEDIM), lambda i: (i, 0))]
        + [part_spec] * NSLICE
        + [
            pl.BlockSpec((NODEDIM, 3 * HDIM), lambda i: (0, 0)),
            pl.BlockSpec((HDIM, 3 * HDIM), lambda i: (0, 0)),
            pl.BlockSpec((1, 3 * HDIM), lambda i: (0, 0)),
            pl.BlockSpec((1, 3 * HDIM), lambda i: (0, 0)),
        ],
        out_specs=pl.BlockSpec((BN, HDIM), lambda i: (i, 0)),
        out_shape=jax.ShapeDtypeStruct((N, HDIM), jnp.float32),
    )(x, *parts_list, wi, whh, bi, bh)


def kernel(x, h, edge_index, edge_attr, W_ih_rel, W_hh_rel, b_ih_rel,
           b_hh_rel, W_ih_node, W_hh_node, b_ih_node, b_hh_node):
    src = edge_index[0].reshape(NSLICE, NW, NCHUNK, C)
    dst = edge_index[1].reshape(NSLICE, NW, NCHUNK, C)
    ea_t = edge_attr.T
    wi_rel = W_ih_rel.T
    whh_rel = W_hh_rel.T
    bi_rel = b_ih_rel.reshape(1, -1)
    bh_rel = b_hh_rel.reshape(1, -1)

    parts_list = []
    for s in range(NSLICE):
        h_src = _sc_gather_kernel(s)(h, src)
        msg = _tc_edge(s, ea_t, h_src,
                       wi_rel, whh_rel, bi_rel, bh_rel)
        parts_list.append(_sc_scatter_kernel(s)(msg, dst))

    return _tc_node(x, parts_list,
                    W_ih_node.T, W_hh_node.T,
                    b_ih_node.reshape(1, -1), b_hh_node.reshape(1, -1))


# speedup vs baseline: 2.6458x; 1.1404x over previous
import functools

import jax
import jax.numpy as jnp
from jax import lax
from jax.experimental import pallas as pl
from jax.experimental.pallas import tpu as pltpu
from jax.experimental.pallas import tpu_sc as plsc

N = 10000
E = 320000
NODEDIM = 128
RELDIM = 16
HDIM = 128

NC = 2
NS = 16
NW = NC * NS
NSLICE = 5
ES = E // NSLICE
EW = ES // NW
C = 80
NCHUNK = EW // C
NPAD = 10240
NSTR = NPAD // NS

_MESH = dict(core_axis_name="c", subcore_axis_name="s",
             num_cores=NC, num_subcores=NS)


@functools.cache
def _sc_gather_kernel(s):
    @functools.partial(
        pl.kernel,
        out_type=jax.ShapeDtypeStruct((ES, HDIM), jnp.float32),
        mesh=plsc.VectorSubcoreMesh(**_MESH),
        scratch_types=[
            pltpu.VMEM((NCHUNK, C), jnp.int32),
            pltpu.VMEM_SHARED((N, HDIM), jnp.float32),
            pltpu.VMEM((C, HDIM), jnp.float32),
            pltpu.VMEM((C, HDIM), jnp.float32),
            pltpu.SemaphoreType.DMA,
            pltpu.SemaphoreType.DMA,
            pltpu.SemaphoreType.DMA,
            pltpu.SemaphoreType.DMA,
        ],
    )
    def _sc_gather(h_hbm, src_hbm, out_hbm, idx_v, h_sh,
                   buf0, buf1, rsem0, rsem1, wsem0, wsem1):
        cid = lax.axis_index("c")
        sid = lax.axis_index("s")
        wid = sid * NC + cid
        pltpu.sync_copy(src_hbm.at[s, wid], idx_v)
        bufs = (buf0, buf1)
        rsems = (rsem0, rsem1)
        wsems = (wsem0, wsem1)

        @pl.when(sid < NS - 1)
        def _():
            pltpu.sync_copy(h_hbm.at[pl.ds(sid * 624, 624)],
                            h_sh.at[pl.ds(sid * 624, 624)])

        @pl.when(sid == NS - 1)
        def _():
            pltpu.sync_copy(h_hbm.at[pl.ds(9360, 640)],
                            h_sh.at[pl.ds(9360, 640)])

        plsc.subcore_barrier()

        pltpu.async_copy(h_sh.at[idx_v.at[0]], buf0, rsem0)

        def body(ci, carry):
            for par in range(2):
                @pl.when(ci % 2 == par)
                def _():
                    nxt = 1 - par

                    @pl.when(ci + 1 < NCHUNK)
                    def _():
                        @pl.when(ci >= 1)
                        def _():
                            pltpu.make_async_copy(
                                bufs[nxt], out_hbm.at[pl.ds(0, C)],
                                wsems[nxt]).wait()

                        pltpu.async_copy(h_sh.at[idx_v.at[ci + 1]],
                                         bufs[nxt], rsems[nxt])

                    pltpu.make_async_copy(h_hbm.at[pl.ds(0, C)],
                                          bufs[par], rsems[par]).wait()
                    pltpu.async_copy(bufs[par],
                                     out_hbm.at[pl.ds(wid * EW + ci * C, C)],
                                     wsems[par])
            return carry

        lax.fori_loop(0, NCHUNK, body, 0)
        for par in range(2):
            pltpu.make_async_copy(bufs[par], out_hbm.at[pl.ds(0, C)],
                                  wsems[par]).wait()

    return _sc_gather


@functools.cache
def _sc_scatter_kernel(s):
    @functools.partial(
        pl.kernel,
        out_type=jax.ShapeDtypeStruct((NC, NPAD, HDIM), jnp.float32),
        mesh=plsc.VectorSubcoreMesh(**_MESH),
        scratch_types=[
            pltpu.VMEM((NCHUNK, C), jnp.int32),
            pltpu.VMEM((C, HDIM), jnp.float32),
            pltpu.VMEM((C, HDIM), jnp.float32),
            pltpu.VMEM((C, HDIM), jnp.float32),
            pltpu.VMEM_SHARED((NPAD, HDIM), jnp.float32),
            pltpu.SemaphoreType.DMA,
            pltpu.SemaphoreType.DMA,
            pltpu.SemaphoreType.DMA,
            pltpu.SemaphoreType.DMA,
        ],
    )
    def _sc_scatter(msg_hbm, dst_hbm, out_hbm,
                    idx_v, buf0, buf1, zbuf, acc_sh,
                    rsem0, rsem1, asem0, asem1):
        cid = lax.axis_index("c")
        sid = lax.axis_index("s")
        wid = sid * NC + cid
        bufs = (buf0, buf1)
        rsems = (rsem0, rsem1)
        asems = (asem0, asem1)

        pltpu.async_copy(msg_hbm.at[pl.ds(wid * EW, C)], buf0, rsem0)
        pltpu.sync_copy(dst_hbm.at[s, wid], idx_v)

        zero16 = jnp.zeros((16,), jnp.float32)

        def zrow(i, carry):
            for j in range(HDIM // 16):
                zbuf[i, pl.ds(j * 16, 16)] = zero16
            return carry

        lax.fori_loop(0, C, zrow, 0)
        for rep in range(NSTR // C):
            pltpu.sync_copy(zbuf, acc_sh.at[pl.ds(sid * NSTR + rep * C, C)])
        plsc.subcore_barrier()

        def body(ci, carry):
            for par in range(2):
                @pl.when(ci % 2 == par)
                def _():
                    nxt = 1 - par

                    @pl.when(ci + 1 < NCHUNK)
                    def _():
                        @pl.when(ci >= 1)
                        def _():
                            pltpu.make_async_copy(
                                bufs[nxt], acc_sh.at[pl.ds(0, C)],
                                asems[nxt]).wait()

                        pltpu.async_copy(
                            msg_hbm.at[pl.ds(wid * EW + (ci + 1) * C, C)],
                            bufs[nxt], rsems[nxt])

                    pltpu.make_async_copy(msg_hbm.at[pl.ds(0, C)],
                                          bufs[par], rsems[par]).wait()
                    pltpu.async_copy(bufs[par], acc_sh.at[idx_v.at[ci]],
                                     asems[par], add=True)
            return carry

        lax.fori_loop(0, NCHUNK, body, 0)
        for par in range(2):
            pltpu.make_async_copy(bufs[par], acc_sh.at[pl.ds(0, C)],
                                  asems[par]).wait()
        plsc.subcore_barrier()
        pltpu.sync_copy(acc_sh.at[pl.ds(sid * NSTR, NSTR)],
                        out_hbm.at[cid, pl.ds(sid * NSTR, NSTR)])

    return _sc_scatter


def _edge_body(ea_ref, hs_ref, wi_ref, whh_ref, bi_ref, bh_ref, out_ref):
    gi = lax.dot_general(ea_ref[...], wi_ref[...],
                         (((0,), (0,)), ((), ())),
                         preferred_element_type=jnp.float32) + bi_ref[...]
    gh = jnp.dot(hs_ref[...], whh_ref[...],
                 preferred_element_type=jnp.float32) + bh_ref[...]
    r = jax.nn.sigmoid(gi[:, :HDIM] + gh[:, :HDIM])
    z = jax.nn.sigmoid(gi[:, HDIM:2 * HDIM] + gh[:, HDIM:2 * HDIM])
    n = jnp.tanh(gi[:, 2 * HDIM:] + r * gh[:, 2 * HDIM:])
    out_ref[...] = (1.0 - z) * n + z * hs_ref[...]


BE = 2560
GE = ES // BE


def _tc_edge(s, edge_attr, h_src, wi, whh, bi, bh):
    return pl.pallas_call(
        _edge_body,
        grid=(GE,),
        in_specs=[
            pl.BlockSpec((RELDIM, BE), lambda i, s=s: (0, i + s * GE)),
            pl.BlockSpec((BE, HDIM), lambda i: (i, 0)),
            pl.BlockSpec((RELDIM, 3 * HDIM), lambda i: (0, 0)),
            pl.BlockSpec((HDIM, 3 * HDIM), lambda i: (0, 0)),
            pl.BlockSpec((1, 3 * HDIM), lambda i: (0, 0)),
            pl.BlockSpec((1, 3 * HDIM), lambda i: (0, 0)),
        ],
        out_specs=pl.BlockSpec((BE, HDIM), lambda i: (i, 0)),
        out_shape=jax.ShapeDtypeStruct((ES, HDIM), jnp.float32),
    )(edge_attr, h_src, wi, whh, bi, bh)


def _node_body(x_ref, *rest):
    parts = rest[:NSLICE]
    wi_ref, whh_ref, bi_ref, bh_ref, out_ref = rest[NSLICE:]
    red = parts[0][0] + parts[0][1]
    for p in parts[1:]:
        red = red + p[0] + p[1]
    gi = jnp.dot(x_ref[...], wi_ref[...],
                 preferred_element_type=jnp.float32) + bi_ref[...]
    gh = jnp.dot(red, whh_ref[...],
                 preferred_element_type=jnp.float32) + bh_ref[...]
    r = jax.nn.sigmoid(gi[:, :HDIM] + gh[:, :HDIM])
    z = jax.nn.sigmoid(gi[:, HDIM:2 * HDIM] + gh[:, HDIM:2 * HDIM])
    n = jnp.tanh(gi[:, 2 * HDIM:] + r * gh[:, 2 * HDIM:])
    out_ref[...] = (1.0 - z) * n + z * red


BN = 1000
GN = N // BN


def _tc_node(x, parts_list, wi, whh, bi, bh):
    part_spec = pl.BlockSpec((NC, BN, HDIM), lambda i: (0, i, 0))
    return pl.pallas_call(
        _node_body,
        grid=(GN,),
        in_specs=[pl.BlockSpec((BN, NODEDIM), lambda i: (i, 0))]
        + [part_spec] * NSLICE
        + [
            pl.BlockSpec((NODEDIM, 3 * HDIM), lambda i: (0, 0)),
            pl.BlockSpec((HDIM, 3 * HDIM), lambda i: (0, 0)),
            pl.BlockSpec((1, 3 * HDIM), lambda i: (0, 0)),
            pl.BlockSpec((1, 3 * HDIM), lambda i: (0, 0)),
        ],
        out_specs=pl.BlockSpec((BN, HDIM), lambda i: (i, 0)),
        out_shape=jax.ShapeDtypeStruct((N, HDIM), jnp.float32),
    )(x, *parts_list, wi, whh, bi, bh)


def kernel(x, h, edge_index, edge_attr, W_ih_rel, W_hh_rel, b_ih_rel,
           b_hh_rel, W_ih_node, W_hh_node, b_ih_node, b_hh_node):
    src = edge_index[0].reshape(NSLICE, NW, NCHUNK, C)
    dst = edge_index[1].reshape(NSLICE, NW, NCHUNK, C)
    ea_t = edge_attr.T
    wi_rel = W_ih_rel.T
    whh_rel = W_hh_rel.T
    bi_rel = b_ih_rel.reshape(1, -1)
    bh_rel = b_hh_rel.reshape(1, -1)

    parts_list = []
    for s in range(NSLICE):
        h_src = _sc_gather_kernel(s)(h, src)
        msg = _tc_edge(s, ea_t, h_src,
                       wi_rel, whh_rel, bi_rel, bh_rel)
        parts_list.append(_sc_scatter_kernel(s)(msg, dst))

    return _tc_node(x, parts_list,
                    W_ih_node.T, W_hh_node.T,
                    b_ih_node.reshape(1, -1), b_hh_node.reshape(1, -1))
